# baseline (speedup 1.0000x reference)
import jax
import jax.numpy as jnp
import numpy as np
from jax import lax
from jax.experimental import pallas as pl
from jax.experimental.pallas import tpu as pltpu

_TWO_PI = 2.0 * np.pi


def _roll(x, s, axis):
    n = x.shape[axis]
    return pltpu.roll(x, s % n, axis)


def _dot(a, b):
    return lax.dot_general(a, b, (((1,), (0,)), ((), ())),
                           precision=lax.Precision.HIGHEST,
                           preferred_element_type=jnp.float32)


def _sincos_turn(u):
    t4 = u * 4.0
    qf = jnp.floor(t4 + 0.5)
    r = u - qf * 0.25
    qi = qf.astype(jnp.int32)
    b0 = (qi & 1) != 0
    b1 = (qi & 2) != 0
    s2 = r * r
    c = 1.0 + s2 * (-19.739208802178716 + s2 * (64.93939402266829
        + s2 * (-85.45681720669371 + s2 * 60.24464137187666)))
    s = r * (6.283185307179586 + s2 * (-41.34170224039975
        + s2 * (81.60524927607504 + s2 * (-76.70585975306136
        + s2 * 42.05869394489765))))
    x = jnp.where(b0, s, c)
    y = jnp.where(b0, c, s)
    cos_v = jnp.where(b0 ^ b1, -x, x)
    sin_v = jnp.where(b1, -y, y)
    return sin_v, cos_v


def _mc_body(th_ref, rpb_ref, rab_ref, rpw_ref, raw_ref, t_ref,
             rpt_ref, p2_ref, pt2_ref, th_out_ref, e_out_ref):
    L, H = rpb_ref.shape[2], rpb_ref.shape[3]
    meven = (lax.broadcasted_iota(jnp.int32, (L, H), 0) % 2) == 0

    results = []
    for j in (0, 1):
        t_j = t_ref[0, 0, 0, j]
        teto = _dot(th_ref[j, 0], p2_ref[...])
        te = teto[:, :H]
        to = teto[:, H:]
        b_u = jnp.where(meven, te, to)
        w_u = jnp.where(meven, to, te)

        def sub_update(cur_u, other_u, prop_u, racc, flip):
            cur = cur_u * _TWO_PI
            prop = prop_u * _TWO_PI
            other = other_u * _TWO_PI
            n_up = _roll(other, 1, 0)
            n_dn = _roll(other, -1, 0)
            o_l = _roll(other, 1, 1)
            o_r = _roll(other, -1, 1)
            if flip:
                n_lf = jnp.where(meven, o_l, other)
                n_rt = jnp.where(meven, other, o_r)
            else:
                n_lf = jnp.where(meven, other, o_l)
                n_rt = jnp.where(meven, o_r, other)
            e_old = -(((jnp.cos(cur - n_up) + jnp.cos(cur - n_dn))
                       + jnp.cos(cur - n_lf)) + jnp.cos(cur - n_rt))
            e_new = -(((jnp.cos(prop - n_up) + jnp.cos(prop - n_dn))
                       + jnp.cos(prop - n_lf)) + jnp.cos(prop - n_rt))
            d_e = e_new - e_old
            acc = racc < jnp.exp(jnp.minimum(-d_e / t_j, 30.0))
            return jnp.where(acc, prop_u, cur_u)

        bn_u = sub_update(b_u, w_u, rpb_ref[j, 0], rab_ref[j, 0], True)
        wn_u = sub_update(w_u, bn_u, rpw_ref[j, 0], raw_ref[j, 0], False)
        bnew = bn_u * _TWO_PI
        wnew = wn_u * _TWO_PI

        sb, cb = _sincos_turn(bn_u)
        sw, cw = _sincos_turn(wn_u)
        bond = (cb * (_roll(cw, -1, 0) + jnp.where(meven, cw, _roll(cw, -1, 1)))
                + sb * (_roll(sw, -1, 0) + jnp.where(meven, sw, _roll(sw, -1, 1)))
                + cw * (_roll(cb, -1, 0) + jnp.where(meven, _roll(cb, -1, 1), cb))
                + sw * (_roll(sb, -1, 0) + jnp.where(meven, _roll(sb, -1, 1), sb)))
        e_img = -jnp.sum(bond)
        ev = jnp.where(meven, bnew, wnew)
        od = jnp.where(meven, wnew, bnew)
        full = _dot(jnp.concatenate([ev, od], axis=1), pt2_ref[...])
        results.append((full, e_img, t_j))

    (full_a, e0, t0), (full_b, e1, t1) = results
    dlt = (1.0 / t0 - 1.0 / t1) * (e0 - e1)
    accp = rpt_ref[0, 0, 0, 0] < jnp.exp(jnp.minimum(dlt, 30.0))
    e_out_ref[0, 0, 0] = jnp.full((8, 128), jnp.where(accp, e1, e0),
                                  dtype=jnp.float32)
    e_out_ref[0, 0, 1] = jnp.full((8, 128), jnp.where(accp, e0, e1),
                                  dtype=jnp.float32)
    th_out_ref[0, 0] = jnp.where(accp, full_b, full_a)
    th_out_ref[1, 0] = jnp.where(accp, full_a, full_b)


@jax.jit
def kernel(theta, T, rand_prop_black, rand_acc_black, rand_prop_white,
           rand_acc_white, rand_pt):
    B, C, L, _ = theta.shape
    H = L // 2
    P = B // 2

    rpb = rand_prop_black.reshape(B, C, L, H)
    rab = rand_acc_black.reshape(B, C, L, H)
    rpw = rand_prop_white.reshape(B, C, L, H)
    raw = rand_acc_white.reshape(B, C, L, H)
    t4 = T.reshape(P, 1, 1, 2)
    eye = np.eye(H, dtype=np.float32)
    pe = np.zeros((L, H), dtype=np.float32)
    po = np.zeros((L, H), dtype=np.float32)
    pe[0::2, :] = eye
    po[1::2, :] = eye
    p2 = np.concatenate([pe, po], axis=1)
    p2_j = jnp.asarray(p2)
    pt2_j = jnp.asarray(p2.T.copy())
    rpt4 = rand_pt.reshape(P, C, 1, 1)

    big = pl.BlockSpec((2, 1, L, H), lambda p, c: (p, c, 0, 0))
    full_spec = pl.BlockSpec((2, 1, L, L), lambda p, c: (p, c, 0, 0))
    t_spec = pl.BlockSpec((1, 1, 1, 2), lambda p, c: (p, 0, 0, 0))
    rpt_spec = pl.BlockSpec((1, 1, 1, 1), lambda p, c: (p, c, 0, 0))
    e_spec = pl.BlockSpec((1, 1, 2, 8, 128), lambda p, c: (p, c, 0, 0, 0))
    p2_spec = pl.BlockSpec((L, L), lambda p, c: (0, 0))

    th_out, e5 = pl.pallas_call(
        _mc_body,
        grid=(P, C),
        in_specs=[full_spec, big, big, big, big, t_spec, rpt_spec,
                  p2_spec, p2_spec],
        out_specs=[full_spec, e_spec],
        out_shape=[
            jax.ShapeDtypeStruct((B, C, L, L), jnp.float32),
            jax.ShapeDtypeStruct((P, C, 2, 8, 128), jnp.float32),
        ],
    )(theta, rpb, rab, rpw, raw, t4, rpt4, p2_j, pt2_j)

    e_out = e5[:, :, :, 0, 0].transpose(0, 2, 1).reshape(B, C)
    return th_out, e_out


# speedup vs baseline: 1.0143x; 1.0143x over previous
import jax
import jax.numpy as jnp
import numpy as np
from jax import lax
from jax.experimental import pallas as pl
from jax.experimental.pallas import tpu as pltpu

_TWO_PI = 2.0 * np.pi


def _roll(x, s, axis):
    n = x.shape[axis]
    return pltpu.roll(x, s % n, axis)


def _dot(a, b):
    return lax.dot_general(a, b, (((1,), (0,)), ((), ())),
                           precision=lax.Precision.HIGHEST,
                           preferred_element_type=jnp.float32)


def _sincos_turn(u):
    t4 = u * 4.0
    qf = jnp.floor(t4 + 0.5)
    r = u - qf * 0.25
    qi = qf.astype(jnp.int32)
    b0 = (qi & 1) != 0
    b1 = (qi & 2) != 0
    s2 = r * r
    c = 1.0 + s2 * (-19.739208802178716 + s2 * (64.93939402266829
        + s2 * (-85.45681720669371 + s2 * 60.24464137187666)))
    s = r * (6.283185307179586 + s2 * (-41.34170224039975
        + s2 * (81.60524927607504 + s2 * (-76.70585975306136
        + s2 * 42.05869394489765))))
    x = jnp.where(b0, s, c)
    y = jnp.where(b0, c, s)
    cos_v = jnp.where(b0 ^ b1, -x, x)
    sin_v = jnp.where(b1, -y, y)
    return sin_v, cos_v


def _mc_body(th_ref, rpb_ref, rab_ref, rpw_ref, raw_ref, t_ref,
             rpt_ref, p2_ref, pt2_ref, th_out_ref, e_out_ref):
    L, H = rpb_ref.shape[2], rpb_ref.shape[3]
    meven = (lax.broadcasted_iota(jnp.int32, (L, H), 0) % 2) == 0

    results = []
    for j in (0, 1):
        t_j = t_ref[0, 0, 0, j]
        teto = _dot(th_ref[j, 0], p2_ref[...])
        te = teto[:, :H]
        to = teto[:, H:]
        b_u = jnp.where(meven, te, to)
        w_u = jnp.where(meven, to, te)

        def sub_update(cur_u, other_u, prop_u, racc, flip):
            cur = cur_u * _TWO_PI
            prop = prop_u * _TWO_PI
            other = other_u * _TWO_PI
            n_up = _roll(other, 1, 0)
            n_dn = _roll(other, -1, 0)
            o_l = _roll(other, 1, 1)
            o_r = _roll(other, -1, 1)
            if flip:
                n_lf = jnp.where(meven, o_l, other)
                n_rt = jnp.where(meven, other, o_r)
            else:
                n_lf = jnp.where(meven, other, o_l)
                n_rt = jnp.where(meven, o_r, other)
            e_old = -(((jnp.cos(cur - n_up) + jnp.cos(cur - n_dn))
                       + jnp.cos(cur - n_lf)) + jnp.cos(cur - n_rt))
            e_new = -(((jnp.cos(prop - n_up) + jnp.cos(prop - n_dn))
                       + jnp.cos(prop - n_lf)) + jnp.cos(prop - n_rt))
            d_e = e_new - e_old
            acc = racc < jnp.exp(jnp.minimum(-d_e / t_j, 30.0))
            return jnp.where(acc, prop_u, cur_u)

        bn_u = sub_update(b_u, w_u, rpb_ref[j, 0], rab_ref[j, 0], True)
        wn_u = sub_update(w_u, bn_u, rpw_ref[j, 0], raw_ref[j, 0], False)
        bnew = bn_u * _TWO_PI
        wnew = wn_u * _TWO_PI

        sb, cb = _sincos_turn(bn_u)
        sw, cw = _sincos_turn(wn_u)
        bond = (cb * (_roll(cw, -1, 0) + jnp.where(meven, cw, _roll(cw, -1, 1)))
                + sb * (_roll(sw, -1, 0) + jnp.where(meven, sw, _roll(sw, -1, 1)))
                + cw * (_roll(cb, -1, 0) + jnp.where(meven, _roll(cb, -1, 1), cb))
                + sw * (_roll(sb, -1, 0) + jnp.where(meven, _roll(sb, -1, 1), sb)))
        e_img = -jnp.sum(bond)
        results.append((bnew, wnew, e_img, t_j))

    planes = []
    for bnew, wnew, _, _ in results:
        ev = jnp.where(meven, bnew, wnew)
        od = jnp.where(meven, wnew, bnew)
        planes.append(jnp.concatenate([ev, od], axis=1))
    cat = jnp.concatenate(planes, axis=0)
    full2 = _dot(cat, pt2_ref[...])
    full_a, full_b = full2[:L], full2[L:]

    (_, _, e0, t0), (_, _, e1, t1) = results
    dlt = (1.0 / t0 - 1.0 / t1) * (e0 - e1)
    accp = rpt_ref[0, 0, 0, 0] < jnp.exp(jnp.minimum(dlt, 30.0))
    e_out_ref[0, 0, 0] = jnp.full((8, 128), jnp.where(accp, e1, e0),
                                  dtype=jnp.float32)
    e_out_ref[0, 0, 1] = jnp.full((8, 128), jnp.where(accp, e0, e1),
                                  dtype=jnp.float32)
    th_out_ref[0, 0] = jnp.where(accp, full_b, full_a)
    th_out_ref[1, 0] = jnp.where(accp, full_a, full_b)


@jax.jit
def kernel(theta, T, rand_prop_black, rand_acc_black, rand_prop_white,
           rand_acc_white, rand_pt):
    B, C, L, _ = theta.shape
    H = L // 2
    P = B // 2

    rpb = rand_prop_black.reshape(B, C, L, H)
    rab = rand_acc_black.reshape(B, C, L, H)
    rpw = rand_prop_white.reshape(B, C, L, H)
    raw = rand_acc_white.reshape(B, C, L, H)
    t4 = T.reshape(P, 1, 1, 2)
    eye = np.eye(H, dtype=np.float32)
    pe = np.zeros((L, H), dtype=np.float32)
    po = np.zeros((L, H), dtype=np.float32)
    pe[0::2, :] = eye
    po[1::2, :] = eye
    p2 = np.concatenate([pe, po], axis=1)
    p2_j = jnp.asarray(p2)
    pt2_j = jnp.asarray(p2.T.copy())
    rpt4 = rand_pt.reshape(P, C, 1, 1)

    big = pl.BlockSpec((2, 1, L, H), lambda p, c: (p, c, 0, 0))
    full_spec = pl.BlockSpec((2, 1, L, L), lambda p, c: (p, c, 0, 0))
    t_spec = pl.BlockSpec((1, 1, 1, 2), lambda p, c: (p, 0, 0, 0))
    rpt_spec = pl.BlockSpec((1, 1, 1, 1), lambda p, c: (p, c, 0, 0))
    e_spec = pl.BlockSpec((1, 1, 2, 8, 128), lambda p, c: (p, c, 0, 0, 0))
    p2_spec = pl.BlockSpec((L, L), lambda p, c: (0, 0))

    th_out, e5 = pl.pallas_call(
        _mc_body,
        grid=(P, C),
        in_specs=[full_spec, big, big, big, big, t_spec, rpt_spec,
                  p2_spec, p2_spec],
        out_specs=[full_spec, e_spec],
        out_shape=[
            jax.ShapeDtypeStruct((B, C, L, L), jnp.float32),
            jax.ShapeDtypeStruct((P, C, 2, 8, 128), jnp.float32),
        ],
    )(theta, rpb, rab, rpw, raw, t4, rpt4, p2_j, pt2_j)

    e_out = e5[:, :, :, 0, 0].transpose(0, 2, 1).reshape(B, C)
    return th_out, e_out


# speedup vs baseline: 1.0588x; 1.0439x over previous
import jax
import jax.numpy as jnp
import numpy as np
from jax import lax
from jax.experimental import pallas as pl
from jax.experimental.pallas import tpu as pltpu

_TWO_PI = 2.0 * np.pi


def _roll(x, s, axis):
    n = x.shape[axis]
    return pltpu.roll(x, s % n, axis)


def _dot(a, b):
    return lax.dot_general(a, b, (((1,), (0,)), ((), ())),
                           precision=lax.Precision.HIGHEST,
                           preferred_element_type=jnp.float32)


def _mc_body(th_ref, rpb_ref, rab_ref, rpw_ref, raw_ref, t_ref,
             rpt_ref, p2_ref, pt2_ref, th_out_ref, e_out_ref):
    L, H = rpb_ref.shape[2], rpb_ref.shape[3]
    meven = (lax.broadcasted_iota(jnp.int32, (L, H), 0) % 2) == 0

    results = []
    for j in (0, 1):
        t_j = t_ref[0, 0, 0, j]
        teto = _dot(th_ref[j, 0], p2_ref[...])
        te = teto[:, :H]
        to = teto[:, H:]
        b_u = jnp.where(meven, te, to)
        w_u = jnp.where(meven, to, te)

        def sub_update(cur_u, other_u, prop_u, racc, flip):
            cur = cur_u * _TWO_PI
            prop = prop_u * _TWO_PI
            other = other_u * _TWO_PI
            n_up = _roll(other, 1, 0)
            n_dn = _roll(other, -1, 0)
            o_l = _roll(other, 1, 1)
            o_r = _roll(other, -1, 1)
            if flip:
                n_lf = jnp.where(meven, o_l, other)
                n_rt = jnp.where(meven, other, o_r)
            else:
                n_lf = jnp.where(meven, other, o_l)
                n_rt = jnp.where(meven, o_r, other)
            e_old = -(((jnp.cos(cur - n_up) + jnp.cos(cur - n_dn))
                       + jnp.cos(cur - n_lf)) + jnp.cos(cur - n_rt))
            e_new = -(((jnp.cos(prop - n_up) + jnp.cos(prop - n_dn))
                       + jnp.cos(prop - n_lf)) + jnp.cos(prop - n_rt))
            d_e = e_new - e_old
            acc = racc < jnp.exp(jnp.minimum(-d_e / t_j, 30.0))
            return jnp.where(acc, prop_u, cur_u), e_old, e_new, acc

        bn_u, _, _, _ = sub_update(b_u, w_u, rpb_ref[j, 0], rab_ref[j, 0],
                                   True)
        wn_u, e_old_w, e_new_w, acc_w = sub_update(
            w_u, bn_u, rpw_ref[j, 0], raw_ref[j, 0], False)
        bnew = bn_u * _TWO_PI
        wnew = wn_u * _TWO_PI

        e_img = jnp.sum(jnp.where(acc_w, e_new_w, e_old_w))
        results.append((bnew, wnew, e_img, t_j))

    planes = []
    for bnew, wnew, _, _ in results:
        ev = jnp.where(meven, bnew, wnew)
        od = jnp.where(meven, wnew, bnew)
        planes.append(jnp.concatenate([ev, od], axis=1))
    cat = jnp.concatenate(planes, axis=0)
    full2 = _dot(cat, pt2_ref[...])
    full_a, full_b = full2[:L], full2[L:]

    (_, _, e0, t0), (_, _, e1, t1) = results
    dlt = (1.0 / t0 - 1.0 / t1) * (e0 - e1)
    accp = rpt_ref[0, 0, 0, 0] < jnp.exp(jnp.minimum(dlt, 30.0))
    e_out_ref[0, 0, 0] = jnp.full((8, 128), jnp.where(accp, e1, e0),
                                  dtype=jnp.float32)
    e_out_ref[0, 0, 1] = jnp.full((8, 128), jnp.where(accp, e0, e1),
                                  dtype=jnp.float32)
    th_out_ref[0, 0] = jnp.where(accp, full_b, full_a)
    th_out_ref[1, 0] = jnp.where(accp, full_a, full_b)


@jax.jit
def kernel(theta, T, rand_prop_black, rand_acc_black, rand_prop_white,
           rand_acc_white, rand_pt):
    B, C, L, _ = theta.shape
    H = L // 2
    P = B // 2

    rpb = rand_prop_black.reshape(B, C, L, H)
    rab = rand_acc_black.reshape(B, C, L, H)
    rpw = rand_prop_white.reshape(B, C, L, H)
    raw = rand_acc_white.reshape(B, C, L, H)
    t4 = T.reshape(P, 1, 1, 2)
    eye = np.eye(H, dtype=np.float32)
    pe = np.zeros((L, H), dtype=np.float32)
    po = np.zeros((L, H), dtype=np.float32)
    pe[0::2, :] = eye
    po[1::2, :] = eye
    p2 = np.concatenate([pe, po], axis=1)
    p2_j = jnp.asarray(p2)
    pt2_j = jnp.asarray(p2.T.copy())
    rpt4 = rand_pt.reshape(P, C, 1, 1)

    big = pl.BlockSpec((2, 1, L, H), lambda p, c: (p, c, 0, 0))
    full_spec = pl.BlockSpec((2, 1, L, L), lambda p, c: (p, c, 0, 0))
    t_spec = pl.BlockSpec((1, 1, 1, 2), lambda p, c: (p, 0, 0, 0))
    rpt_spec = pl.BlockSpec((1, 1, 1, 1), lambda p, c: (p, c, 0, 0))
    e_spec = pl.BlockSpec((1, 1, 2, 8, 128), lambda p, c: (p, c, 0, 0, 0))
    p2_spec = pl.BlockSpec((L, L), lambda p, c: (0, 0))

    th_out, e5 = pl.pallas_call(
        _mc_body,
        grid=(P, C),
        in_specs=[full_spec, big, big, big, big, t_spec, rpt_spec,
                  p2_spec, p2_spec],
        out_specs=[full_spec, e_spec],
        out_shape=[
            jax.ShapeDtypeStruct((B, C, L, L), jnp.float32),
            jax.ShapeDtypeStruct((P, C, 2, 8, 128), jnp.float32),
        ],
    )(theta, rpb, rab, rpw, raw, t4, rpt4, p2_j, pt2_j)

    e_out = e5[:, :, :, 0, 0].transpose(0, 2, 1).reshape(B, C)
    return th_out, e_out
